# baseline (speedup 1.0000x reference)
import functools

import numpy as np
import jax
import jax.numpy as jnp
from jax import lax
from jax.experimental import pallas as pl
from jax.experimental.pallas import tpu as pltpu
from jax.experimental.pallas import tpu_sc as plsc

_N_FIELDS = 26
_VOCAB = 1000
_EMB = 31
_B = 1024
_L = 20

_NF_PAD = 28
_EMB_PAD = 32
_BL = _B * _L
_NW = 32
_BL_W = _BL // _NW
_SRPB = _NF_PAD // 4
_SR = _BL * _SRPB
_SR_W = _SR // _NW

_CBL = 16
_NIDX = _CBL * _SRPB
_NCH = _BL_W // _CBL

_TBL = 512
_TSR = _TBL * _SRPB
_NBLK = _BL // _TBL
_BAND = 4 * _EMB
_OW = _N_FIELDS * _EMB


def _gather_body(tab_hbm, gidx_hbm, out_hbm, idx_v, stage_v, gsem, wsem0, wsem1):
    wid = lax.axis_index("s") * 2 + lax.axis_index("c")
    row0 = wid * _SR_W

    pltpu.sync_copy(gidx_hbm.at[wid], idx_v)

    wsems = (wsem0, wsem1)

    def gather_chunk(ch, buf):
        cps = [
            pltpu.async_copy(tab_hbm.at[idx_v.at[ch, c]], stage_v.at[buf, c], gsem)
            for c in range(4)
        ]
        for cp in cps:
            cp.wait()

    def wb_descr(ch, buf, c):
        return pltpu.make_async_copy(
            stage_v.at[buf, c],
            out_hbm.at[pl.ds(row0 + ch * _NIDX, _NIDX), pl.ds(32 * c, 32)],
            wsems[buf],
        )

    def writeback(ch, buf):
        for c in range(4):
            wb_descr(ch, buf, c).start()

    def drain_wb(ch, buf):
        for c in range(4):
            wb_descr(ch, buf, c).wait()

    for b in range(2):
        gather_chunk(b, b)
        writeback(b, b)

    @pl.loop(0, _NCH - 2, step=2)
    def _(c0):
        for b in range(2):
            ch = c0 + 2 + b
            drain_wb(ch, b)
            gather_chunk(ch, b)
            writeback(ch, b)

    for b in range(2):
        drain_wb(_NCH - 2 + b, b)


def _compact_body(in_ref, m_ref, out_ref):
    x = in_ref[...].reshape(_TBL, _SRPB, 128)
    ys = [
        jnp.dot(x[:, j, :], m_ref[j], preferred_element_type=jnp.float32)
        for j in range(_SRPB)
    ]
    out_ref[...] = jnp.concatenate(ys, axis=1)[:, :_OW]


def _band_matrices():
    m = np.zeros((_SRPB, 128, _BAND), np.float32)
    for j in range(_SRPB):
        for k in range(128):
            f = 4 * j + k // _EMB_PAD
            e = k % _EMB_PAD
            if f < _N_FIELDS and e < _EMB:
                m[j, k, (f - 4 * j) * _EMB + e] = 1.0
    return m


_M3 = jnp.asarray(_band_matrices())


@jax.jit
def _embed(padded_tables, gidx):
    mesh = plsc.VectorSubcoreMesh(core_axis_name="c", subcore_axis_name="s")
    gather = pl.kernel(
        _gather_body,
        out_type=jax.ShapeDtypeStruct((_SR, 128), jnp.float32),
        mesh=mesh,
        scratch_types=[
            pltpu.VMEM((_NCH, 4, _NIDX), jnp.int32),
            pltpu.VMEM((2, 4, _NIDX, _EMB_PAD), jnp.float32),
            pltpu.SemaphoreType.DMA,
            pltpu.SemaphoreType.DMA,
            pltpu.SemaphoreType.DMA,
        ],
        compiler_params=pltpu.CompilerParams(use_tc_tiling_on_sc=False),
    )
    inter = gather(padded_tables, gidx)

    compact = pl.pallas_call(
        _compact_body,
        grid=(_NBLK,),
        in_specs=[
            pl.BlockSpec((_TSR, 128), lambda i: (i, 0)),
            pl.BlockSpec((_SRPB, 128, _BAND), lambda i: (0, 0, 0)),
        ],
        out_specs=pl.BlockSpec((_TBL, _OW), lambda i: (i, 0)),
        out_shape=jax.ShapeDtypeStruct((_BL, _OW), jnp.float32),
    )
    return compact(inter, _M3)


def kernel(indices, tables):
    flat_tables = tables.reshape(_N_FIELDS * _VOCAB, _EMB)
    padded_tables = jnp.pad(flat_tables, ((0, 0), (0, _EMB_PAD - _EMB)))

    offs = jnp.arange(_N_FIELDS, dtype=jnp.int32) * _VOCAB
    g = (indices.astype(jnp.int32) + offs).reshape(_BL, _N_FIELDS)
    g = jnp.pad(g, ((0, 0), (0, _NF_PAD - _N_FIELDS)))
    g = g.reshape(_NW, _NCH, _CBL, _SRPB, 4).transpose(0, 1, 4, 2, 3)
    gidx = g.reshape(_NW, _NCH, 4, _NIDX)

    out = _embed(padded_tables, gidx)
    return out.reshape(_B, _L, _OW)


# speedup vs baseline: 1.6276x; 1.6276x over previous
import functools

import jax
import jax.numpy as jnp
from jax import lax
from jax.experimental import pallas as pl
from jax.experimental.pallas import tpu as pltpu
from jax.experimental.pallas import tpu_sc as plsc

_N_FIELDS = 26
_VOCAB = 1000
_EMB = 31
_B = 1024
_L = 20

_EMB_PAD = 32
_NW = 32
_ROWS = _B * _L * _N_FIELDS
_RPW = _ROWS // _NW
_IW = 128
_G = 5
_CHUNK = _G * _IW
_NCH = _RPW // _CHUNK
_IDX_ROWS = _RPW // _IW


def _emb_body(tab_hbm, gidx_hbm, out_hbm, idx_v, rows_v, gsem, wsem0, wsem1):
    wid = lax.axis_index("s") * 2 + lax.axis_index("c")
    row0 = wid * _RPW

    pltpu.sync_copy(gidx_hbm.at[wid], idx_v)

    wsems = (wsem0, wsem1)

    def gather_chunk(ch, buf):
        cps = [
            pltpu.async_copy(
                tab_hbm.at[idx_v.at[ch * _G + g]],
                rows_v.at[buf, pl.ds(g * _IW, _IW), :],
                gsem,
            )
            for g in range(_G)
        ]
        for cp in cps:
            cp.wait()

    def wb_descr(ch, buf):
        return pltpu.make_async_copy(
            rows_v.at[buf],
            out_hbm.at[pl.ds(row0 + ch * _CHUNK, _CHUNK), :],
            wsems[buf],
        )

    for b in range(2):
        gather_chunk(b, b)
        wb_descr(b, b).start()

    @pl.loop(0, _NCH - 2, step=2)
    def _(c0):
        for b in range(2):
            ch = c0 + 2 + b
            wb_descr(ch, b).wait()
            gather_chunk(ch, b)
            wb_descr(ch, b).start()

    for b in range(2):
        wb_descr(_NCH - 2 + b, b).wait()


@jax.jit
def _embed(padded_tables, gidx):
    mesh = plsc.VectorSubcoreMesh(core_axis_name="c", subcore_axis_name="s")
    run = pl.kernel(
        _emb_body,
        out_type=jax.ShapeDtypeStruct((_ROWS, _EMB_PAD), jnp.float32),
        mesh=mesh,
        scratch_types=[
            pltpu.VMEM((_IDX_ROWS, _IW), jnp.int32),
            pltpu.VMEM((2, _CHUNK, _EMB_PAD), jnp.float32),
            pltpu.SemaphoreType.DMA,
            pltpu.SemaphoreType.DMA,
            pltpu.SemaphoreType.DMA,
        ],
        compiler_params=pltpu.CompilerParams(use_tc_tiling_on_sc=False),
    )
    return run(padded_tables, gidx)


def kernel(indices, tables):
    flat_tables = tables.reshape(_N_FIELDS * _VOCAB, _EMB)
    padded_tables = jnp.pad(flat_tables, ((0, 0), (0, _EMB_PAD - _EMB)))
    offs = jnp.arange(_N_FIELDS, dtype=jnp.int32) * _VOCAB
    gidx = (indices.astype(jnp.int32) + offs).reshape(_NW, _IDX_ROWS, _IW)
    out = _embed(padded_tables, gidx)
    out = out[:, :_EMB]
    return out.reshape(_B, _L, _N_FIELDS * _EMB)
